# baseline (speedup 1.0000x reference)
import functools

import jax
import jax.numpy as jnp
from jax import lax
from jax.experimental import pallas as pl
from jax.experimental.pallas import tpu as pltpu
from jax.experimental.pallas import tpu_sc as plsc

_NC, _NS, _L = 2, 16, 16
_NW = _NC * _NS
_CHUNK = 16384


def _sc_body(xf, pef, of, xb0, pb0, ob0, xb1, pb1, ob1, rx0, rp0, w0, rx1, rp1, w1):
    epw = xf.shape[0] // _NW
    nch = epw // _CHUNK
    wid = lax.axis_index("s") * _NC + lax.axis_index("c")
    base = wid * epw

    X = (xb0, xb1)
    P = (pb0, pb1)
    O = (ob0, ob1)
    RX = (rx0, rx1)
    RP = (rp0, rp1)
    W = (w0, w1)

    def rd(ci):
        sl = ci & 1
        off = base + ci * _CHUNK
        pltpu.make_async_copy(xf.at[pl.ds(off, _CHUNK)], X[sl], RX[sl]).start()
        pltpu.make_async_copy(pef.at[pl.ds(off, _CHUNK)], P[sl], RP[sl]).start()

    rd(0)
    if nch > 1:
        rd(1)
    for ci in range(nch):
        sl = ci & 1
        off = base + ci * _CHUNK
        pltpu.make_async_copy(xf.at[pl.ds(off, _CHUNK)], X[sl], RX[sl]).wait()
        pltpu.make_async_copy(pef.at[pl.ds(off, _CHUNK)], P[sl], RP[sl]).wait()
        if ci >= 2:
            poff = base + (ci - 2) * _CHUNK
            pltpu.make_async_copy(O[sl], of.at[pl.ds(poff, _CHUNK)], W[sl]).wait()
        xb, pb, ob = X[sl], P[sl], O[sl]

        @plsc.parallel_loop(0, _CHUNK, step=_L, unroll=8)
        def _(i):
            ob[pl.ds(i, _L)] = xb[pl.ds(i, _L)] + pb[pl.ds(i, _L)]

        pltpu.make_async_copy(O[sl], of.at[pl.ds(off, _CHUNK)], W[sl]).start()
        if ci + 2 < nch:
            rd(ci + 2)
    for ci in range(max(nch - 2, 0), nch):
        sl = ci & 1
        off = base + ci * _CHUNK
        pltpu.make_async_copy(O[sl], of.at[pl.ds(off, _CHUNK)], W[sl]).wait()


@jax.jit
def _pe_add(x, position_embeddings):
    seq_len, d_model = x.shape
    n = seq_len * d_model
    xf = x.reshape(n)
    pef = position_embeddings.reshape(n)
    mesh = plsc.VectorSubcoreMesh(core_axis_name="c", subcore_axis_name="s")
    vmem = lambda: pltpu.VMEM((_CHUNK,), jnp.float32)
    sem = pltpu.SemaphoreType.DMA
    run = functools.partial(
        pl.kernel,
        out_type=jax.ShapeDtypeStruct((n,), jnp.float32),
        mesh=mesh,
        scratch_types=[
            vmem(), vmem(), vmem(), vmem(), vmem(), vmem(),
            sem, sem, sem, sem, sem, sem,
        ],
    )(_sc_body)
    return run(xf, pef).reshape(seq_len, d_model)


def kernel(x, position_embeddings):
    seq_len = x.shape[0]
    return _pe_add(x, position_embeddings[:seq_len])


# speedup vs baseline: 4.6636x; 4.6636x over previous
import jax
import jax.numpy as jnp
from jax.experimental import pallas as pl
from jax.experimental.pallas import tpu as pltpu

_BR = 512
_NBUF = 4


def _body(x_hbm, pe_hbm, o_hbm, xb, pb, ob, rsx, rsp, ws):
    nblk = x_hbm.shape[0] // _BR

    def read(i):
        s = i % _NBUF
        pltpu.make_async_copy(x_hbm.at[pl.ds(i * _BR, _BR)], xb.at[s], rsx.at[s]).start()
        pltpu.make_async_copy(pe_hbm.at[pl.ds(i * _BR, _BR)], pb.at[s], rsp.at[s]).start()

    for i in range(_NBUF):
        read(i)
    for i in range(nblk):
        s = i % _NBUF
        pltpu.make_async_copy(x_hbm.at[pl.ds(i * _BR, _BR)], xb.at[s], rsx.at[s]).wait()
        pltpu.make_async_copy(pe_hbm.at[pl.ds(i * _BR, _BR)], pb.at[s], rsp.at[s]).wait()
        if i >= _NBUF:
            j = i - _NBUF
            pltpu.make_async_copy(ob.at[s], o_hbm.at[pl.ds(j * _BR, _BR)], ws.at[s]).wait()
        ob[s] = xb[s] + pb[s]
        pltpu.make_async_copy(ob.at[s], o_hbm.at[pl.ds(i * _BR, _BR)], ws.at[s]).start()
        if i + _NBUF < nblk:
            read(i + _NBUF)
    for i in range(nblk - _NBUF, nblk):
        s = i % _NBUF
        pltpu.make_async_copy(ob.at[s], o_hbm.at[pl.ds(i * _BR, _BR)], ws.at[s]).wait()


@jax.jit
def _pe_add(x, position_embeddings):
    seq_len, d_model = x.shape
    return pl.pallas_call(
        _body,
        in_specs=[
            pl.BlockSpec(memory_space=pltpu.MemorySpace.HBM),
            pl.BlockSpec(memory_space=pltpu.MemorySpace.HBM),
        ],
        out_specs=pl.BlockSpec(memory_space=pltpu.MemorySpace.HBM),
        out_shape=jax.ShapeDtypeStruct((seq_len, d_model), x.dtype),
        scratch_shapes=[
            pltpu.VMEM((_NBUF, _BR, d_model), jnp.float32),
            pltpu.VMEM((_NBUF, _BR, d_model), jnp.float32),
            pltpu.VMEM((_NBUF, _BR, d_model), jnp.float32),
            pltpu.SemaphoreType.DMA((_NBUF,)),
            pltpu.SemaphoreType.DMA((_NBUF,)),
            pltpu.SemaphoreType.DMA((_NBUF,)),
        ],
    )(x, position_embeddings)


def kernel(x, position_embeddings):
    seq_len = x.shape[0]
    return _pe_add(x, position_embeddings[:seq_len])
